# Initial kernel scaffold; baseline (speedup 1.0000x reference)
import jax
import jax.numpy as jnp
from jax.experimental import pallas as pl


def kernel(x, edge_index, W_lin, b_lin, W1_self, W1_neigh, b1, W2_self, W2_neigh, b2):
    raise NotImplementedError("write your pallas kernel here")


# speedup vs baseline: 16.6184x; 16.6184x over previous
import functools

import jax
import jax.numpy as jnp
from jax import lax
from jax.experimental import pallas as pl
from jax.experimental.pallas import tpu as pltpu
from jax.experimental.pallas import tpu_sc as plsc

N = 10000
E = 320000
D = 128
H = 16
C = 128

NC = 2
NS = 16
NW = NC * NS
CHUNK = 128
CH = 80
EPW = CH * CHUNK
E_PAD = NW * EPW
N_PAD = 10240
RPT = N_PAD // NS

_BN = 2048
_GRID = N_PAD // _BN


def _tc1_body(x_ref, wl_ref, bl_ref, ws_ref, wn_ref, s_ref, q_ref):
    h = jnp.dot(x_ref[...], wl_ref[...], preferred_element_type=jnp.float32)
    h = h + bl_ref[...][None, :]
    s_ref[...] = jnp.dot(h, ws_ref[...], preferred_element_type=jnp.float32)
    q_ref[...] = jnp.dot(h, wn_ref[...], preferred_element_type=jnp.float32)


_tc1 = pl.pallas_call(
    _tc1_body,
    grid=(_GRID,),
    in_specs=[
        pl.BlockSpec((_BN, D), lambda i: (i, 0)),
        pl.BlockSpec((D, C), lambda i: (0, 0)),
        pl.BlockSpec((C,), lambda i: (0,)),
        pl.BlockSpec((C, H), lambda i: (0, 0)),
        pl.BlockSpec((C, H), lambda i: (0, 0)),
    ],
    out_specs=[
        pl.BlockSpec((_BN, H), lambda i: (i, 0)),
        pl.BlockSpec((_BN, H), lambda i: (i, 0)),
    ],
    out_shape=[
        jax.ShapeDtypeStruct((N_PAD, H), jnp.float32),
        jax.ShapeDtypeStruct((N_PAD, H), jnp.float32),
    ],
)


def _make_sc_pass(with_deg: bool):
    out_type = [jax.ShapeDtypeStruct((NC, N_PAD, H), jnp.float32)]
    scratch = [
        pltpu.VMEM((CH, CHUNK), jnp.int32),
        pltpu.VMEM((CH, CHUNK), jnp.int32),
        pltpu.VMEM((2, CHUNK, H), jnp.float32),
        pltpu.VMEM_SHARED((N_PAD, H), jnp.float32),
        pltpu.SemaphoreType.DMA,
        pltpu.SemaphoreType.DMA,
    ]
    if with_deg:
        out_type.append(jax.ShapeDtypeStruct((NC, N_PAD), jnp.float32))
        scratch.append(pltpu.VMEM((CHUNK,), jnp.float32))
        scratch.append(pltpu.VMEM_SHARED((N_PAD,), jnp.float32))

    def body(table, src_idx, dst_idx, z16, zd, *rest):
        if with_deg:
            acc_out, deg_out, srcv, dstv, rows, acc_sh, sem0, sem1, ones_v, deg_sh = rest
        else:
            acc_out, srcv, dstv, rows, acc_sh, sem0, sem1 = rest
        sems = (sem0, sem1)
        cid = lax.axis_index("c")
        sid = lax.axis_index("s")
        wid = sid * NC + cid
        sl = pl.ds(sid * RPT, RPT)

        pltpu.sync_copy(src_idx.at[wid], srcv)
        pltpu.sync_copy(dst_idx.at[wid], dstv)
        pltpu.sync_copy(z16.at[sl], acc_sh.at[sl])
        if with_deg:
            pltpu.sync_copy(zd.at[sl], deg_sh.at[sl])
            for i in range(CHUNK // 16):
                ones_v[pl.ds(i * 16, 16)] = jnp.full((16,), 1.0, jnp.float32)
        plsc.subcore_barrier()

        pltpu.async_copy(table.at[srcv.at[0]], rows.at[0], sems[0])

        def step(i, carry):
            for b in (0, 1):
                c = 2 * i + b
                nb = 1 - b

                @pl.when(c + 1 < CH)
                def _():
                    pltpu.async_copy(table.at[srcv.at[c + 1]], rows.at[nb], sems[nb])

                pltpu.make_async_copy(table.at[srcv.at[c]], rows.at[b], sems[b]).wait()
                pltpu.sync_copy(rows.at[b], acc_sh.at[dstv.at[c]], add=True)
                if with_deg:
                    pltpu.sync_copy(ones_v, deg_sh.at[dstv.at[c]], add=True)
            return carry

        lax.fori_loop(0, CH // 2, step, 0)

        plsc.subcore_barrier()
        pltpu.sync_copy(acc_sh.at[sl], acc_out.at[cid, sl])
        if with_deg:
            pltpu.sync_copy(deg_sh.at[sl], deg_out.at[cid, sl])

    return pl.kernel(
        body,
        out_type=out_type,
        mesh=plsc.VectorSubcoreMesh(core_axis_name="c", subcore_axis_name="s"),
        scratch_types=scratch,
        compiler_params=pltpu.CompilerParams(use_tc_tiling_on_sc=False),
    )


_sc_pass_deg = _make_sc_pass(True)
_sc_pass = _make_sc_pass(False)


def _tc2_body(s_ref, acc_ref, deg_ref, b1_ref, h1_ref):
    a = acc_ref[0] + acc_ref[1]
    dg = deg_ref[0] + deg_ref[1]
    maxd = jnp.maximum(dg, 1.0)[:, None]
    h1_ref[...] = jnp.tanh(s_ref[...] + b1_ref[...][None, :] + a / maxd)


_tc2 = pl.pallas_call(
    _tc2_body,
    grid=(_GRID,),
    in_specs=[
        pl.BlockSpec((_BN, H), lambda i: (i, 0)),
        pl.BlockSpec((NC, _BN, H), lambda i: (0, i, 0)),
        pl.BlockSpec((NC, _BN), lambda i: (0, i)),
        pl.BlockSpec((H,), lambda i: (0,)),
    ],
    out_specs=pl.BlockSpec((_BN, H), lambda i: (i, 0)),
    out_shape=jax.ShapeDtypeStruct((N_PAD, H), jnp.float32),
)


def _tc3_body(h1_ref, acc_ref, deg_ref, ws_ref, wn_ref, b2_ref, out_ref):
    a = acc_ref[0] + acc_ref[1]
    dg = deg_ref[0] + deg_ref[1]
    agg = a / jnp.maximum(dg, 1.0)[:, None]
    out_ref[...] = jnp.tanh(
        jnp.dot(h1_ref[...], ws_ref[...], preferred_element_type=jnp.float32)
        + jnp.dot(agg, wn_ref[...], preferred_element_type=jnp.float32)
        + b2_ref[...][None, :]
    )


_tc3 = pl.pallas_call(
    _tc3_body,
    grid=(_GRID,),
    in_specs=[
        pl.BlockSpec((_BN, H), lambda i: (i, 0)),
        pl.BlockSpec((NC, _BN, H), lambda i: (0, i, 0)),
        pl.BlockSpec((NC, _BN), lambda i: (0, i)),
        pl.BlockSpec((H, C), lambda i: (0, 0)),
        pl.BlockSpec((H, C), lambda i: (0, 0)),
        pl.BlockSpec((C,), lambda i: (0,)),
    ],
    out_specs=pl.BlockSpec((_BN, C), lambda i: (i, 0)),
    out_shape=jax.ShapeDtypeStruct((N_PAD, C), jnp.float32),
)


def kernel(x, edge_index, W_lin, b_lin, W1_self, W1_neigh, b1, W2_self, W2_neigh, b2):
    x_pad = jnp.pad(x, ((0, N_PAD - N), (0, 0)))
    src = jnp.pad(edge_index[0], (0, E_PAD - E), constant_values=N)
    dst = jnp.pad(edge_index[1], (0, E_PAD - E), constant_values=N)
    src_r = src.reshape(NW, CH, CHUNK)
    dst_r = dst.reshape(NW, CH, CHUNK)
    z16 = jnp.zeros((N_PAD, H), jnp.float32)
    zd = jnp.zeros((N_PAD,), jnp.float32)

    s_tab, q_tab = _tc1(x_pad, W_lin, b_lin, W1_self, W1_neigh)
    acc1, deg = _sc_pass_deg(q_tab, src_r, dst_r, z16, zd)
    h1 = _tc2(s_tab, acc1, deg, b1)
    (acc2,) = _sc_pass(h1, src_r, dst_r, z16, zd)
    out = _tc3(h1, acc2, deg, W2_self, W2_neigh, b2)
    return out[:N]


# speedup vs baseline: 17.3788x; 1.0458x over previous
import functools

import jax
import jax.numpy as jnp
from jax import lax
from jax.experimental import pallas as pl
from jax.experimental.pallas import tpu as pltpu
from jax.experimental.pallas import tpu_sc as plsc

N = 10000
E = 320000
D = 128
H = 16
C = 128

NC = 2
NS = 16
NW = NC * NS
CHUNK = 128
CH = 80
EPW = CH * CHUNK
E_PAD = NW * EPW
N_PAD = 10240
RPT = N_PAD // NS

_BN = 2048
_GRID = N_PAD // _BN


def _tc1_body(x_ref, wl_ref, bl_ref, ws_ref, wn_ref, s_ref, q_ref):
    h = jnp.dot(x_ref[...], wl_ref[...], preferred_element_type=jnp.float32)
    h = h + bl_ref[...][None, :]
    s_ref[...] = jnp.dot(h, ws_ref[...], preferred_element_type=jnp.float32)
    q_ref[...] = jnp.dot(h, wn_ref[...], preferred_element_type=jnp.float32)


_tc1 = pl.pallas_call(
    _tc1_body,
    grid=(_GRID,),
    in_specs=[
        pl.BlockSpec((_BN, D), lambda i: (i, 0)),
        pl.BlockSpec((D, C), lambda i: (0, 0)),
        pl.BlockSpec((C,), lambda i: (0,)),
        pl.BlockSpec((C, H), lambda i: (0, 0)),
        pl.BlockSpec((C, H), lambda i: (0, 0)),
    ],
    out_specs=[
        pl.BlockSpec((_BN, H), lambda i: (i, 0)),
        pl.BlockSpec((_BN, H), lambda i: (i, 0)),
    ],
    out_shape=[
        jax.ShapeDtypeStruct((N_PAD, H), jnp.float32),
        jax.ShapeDtypeStruct((N_PAD, H), jnp.float32),
    ],
)


NBUF = 8
PREF = 4


def _make_sc_pass(with_deg: bool):
    out_type = [jax.ShapeDtypeStruct((NC, N_PAD, H), jnp.float32)]
    scratch = [
        pltpu.VMEM((CH, CHUNK), jnp.int32),
        pltpu.VMEM((CH, CHUNK), jnp.int32),
        pltpu.VMEM((NBUF, CHUNK, H), jnp.float32),
        pltpu.VMEM_SHARED((N_PAD, H), jnp.float32),
        [pltpu.SemaphoreType.DMA] * NBUF,
        [pltpu.SemaphoreType.DMA] * NBUF,
    ]
    if with_deg:
        out_type.append(jax.ShapeDtypeStruct((NC, N_PAD), jnp.float32))
        scratch.append(pltpu.VMEM((CHUNK,), jnp.float32))
        scratch.append(pltpu.VMEM_SHARED((N_PAD,), jnp.float32))

    def body(table, src_idx, dst_idx, z16, zd, *rest):
        if with_deg:
            acc_out, deg_out, srcv, dstv, rows, acc_sh, sem_g, sem_s, ones_v, deg_sh = rest
        else:
            acc_out, srcv, dstv, rows, acc_sh, sem_g, sem_s = rest
        cid = lax.axis_index("c")
        sid = lax.axis_index("s")
        wid = sid * NC + cid
        sl = pl.ds(sid * RPT, RPT)

        pltpu.sync_copy(src_idx.at[wid], srcv)
        pltpu.sync_copy(dst_idx.at[wid], dstv)
        pltpu.sync_copy(z16.at[sl], acc_sh.at[sl])
        if with_deg:
            pltpu.sync_copy(zd.at[sl], deg_sh.at[sl])
            for i in range(CHUNK // 16):
                ones_v[pl.ds(i * 16, 16)] = jnp.full((16,), 1.0, jnp.float32)
        plsc.subcore_barrier()

        def wait_scatter(b, c):
            pltpu.make_async_copy(rows.at[b], acc_sh.at[dstv.at[c]], sem_s[b]).wait()
            if with_deg:
                pltpu.make_async_copy(ones_v, deg_sh.at[dstv.at[c]], sem_s[b]).wait()

        for b in range(PREF):
            pltpu.async_copy(table.at[srcv.at[b]], rows.at[b], sem_g[b])

        def step(i, carry):
            for b8 in range(NBUF):
                c = NBUF * i + b8
                gb = (b8 + PREF) % NBUF

                @pl.when(c + PREF < CH)
                def _():
                    @pl.when(c >= PREF)
                    def _():
                        wait_scatter(gb, c - PREF)

                    pltpu.async_copy(table.at[srcv.at[c + PREF]], rows.at[gb], sem_g[gb])

                pltpu.make_async_copy(table.at[srcv.at[c]], rows.at[b8], sem_g[b8]).wait()
                pltpu.async_copy(rows.at[b8], acc_sh.at[dstv.at[c]], sem_s[b8], add=True)
                if with_deg:
                    pltpu.async_copy(ones_v, deg_sh.at[dstv.at[c]], sem_s[b8], add=True)
            return carry

        lax.fori_loop(0, CH // NBUF, step, 0)
        for b in range(NBUF):
            wait_scatter(b, CH - NBUF + b)

        plsc.subcore_barrier()
        pltpu.sync_copy(acc_sh.at[sl], acc_out.at[cid, sl])
        if with_deg:
            pltpu.sync_copy(deg_sh.at[sl], deg_out.at[cid, sl])

    return pl.kernel(
        body,
        out_type=out_type,
        mesh=plsc.VectorSubcoreMesh(core_axis_name="c", subcore_axis_name="s"),
        scratch_types=scratch,
        compiler_params=pltpu.CompilerParams(use_tc_tiling_on_sc=False),
    )


_sc_pass_deg = _make_sc_pass(True)
_sc_pass = _make_sc_pass(False)


def _tc2_body(s_ref, acc_ref, deg_ref, b1_ref, h1_ref):
    a = acc_ref[0] + acc_ref[1]
    dg = deg_ref[0] + deg_ref[1]
    maxd = jnp.maximum(dg, 1.0)[:, None]
    h1_ref[...] = jnp.tanh(s_ref[...] + b1_ref[...][None, :] + a / maxd)


_tc2 = pl.pallas_call(
    _tc2_body,
    grid=(_GRID,),
    in_specs=[
        pl.BlockSpec((_BN, H), lambda i: (i, 0)),
        pl.BlockSpec((NC, _BN, H), lambda i: (0, i, 0)),
        pl.BlockSpec((NC, _BN), lambda i: (0, i)),
        pl.BlockSpec((H,), lambda i: (0,)),
    ],
    out_specs=pl.BlockSpec((_BN, H), lambda i: (i, 0)),
    out_shape=jax.ShapeDtypeStruct((N_PAD, H), jnp.float32),
)


def _tc3_body(h1_ref, acc_ref, deg_ref, ws_ref, wn_ref, b2_ref, out_ref):
    a = acc_ref[0] + acc_ref[1]
    dg = deg_ref[0] + deg_ref[1]
    agg = a / jnp.maximum(dg, 1.0)[:, None]
    out_ref[...] = jnp.tanh(
        jnp.dot(h1_ref[...], ws_ref[...], preferred_element_type=jnp.float32)
        + jnp.dot(agg, wn_ref[...], preferred_element_type=jnp.float32)
        + b2_ref[...][None, :]
    )


_tc3 = pl.pallas_call(
    _tc3_body,
    grid=(_GRID,),
    in_specs=[
        pl.BlockSpec((_BN, H), lambda i: (i, 0)),
        pl.BlockSpec((NC, _BN, H), lambda i: (0, i, 0)),
        pl.BlockSpec((NC, _BN), lambda i: (0, i)),
        pl.BlockSpec((H, C), lambda i: (0, 0)),
        pl.BlockSpec((H, C), lambda i: (0, 0)),
        pl.BlockSpec((C,), lambda i: (0,)),
    ],
    out_specs=pl.BlockSpec((_BN, C), lambda i: (i, 0)),
    out_shape=jax.ShapeDtypeStruct((N_PAD, C), jnp.float32),
)


def kernel(x, edge_index, W_lin, b_lin, W1_self, W1_neigh, b1, W2_self, W2_neigh, b2):
    x_pad = jnp.pad(x, ((0, N_PAD - N), (0, 0)))
    src = jnp.pad(edge_index[0], (0, E_PAD - E), constant_values=N)
    dst = jnp.pad(edge_index[1], (0, E_PAD - E), constant_values=N)
    src_r = src.reshape(NW, CH, CHUNK)
    dst_r = dst.reshape(NW, CH, CHUNK)
    z16 = jnp.zeros((N_PAD, H), jnp.float32)
    zd = jnp.zeros((N_PAD,), jnp.float32)

    s_tab, q_tab = _tc1(x_pad, W_lin, b_lin, W1_self, W1_neigh)
    acc1, deg = _sc_pass_deg(q_tab, src_r, dst_r, z16, zd)
    h1 = _tc2(s_tab, acc1, deg, b1)
    (acc2,) = _sc_pass(h1, src_r, dst_r, z16, zd)
    out = _tc3(h1, acc2, deg, W2_self, W2_neigh, b2)
    return out[:N]


# speedup vs baseline: 18.4031x; 1.0589x over previous
import functools

import jax
import jax.numpy as jnp
from jax import lax
from jax.experimental import pallas as pl
from jax.experimental.pallas import tpu as pltpu
from jax.experimental.pallas import tpu_sc as plsc

N = 10000
E = 320000
D = 128
H = 16
C = 128

NC = 2
NS = 16
NW = NC * NS
CHUNK = 128
CH = 80
EPW = CH * CHUNK
E_PAD = NW * EPW
N_PAD = 10240
RPT = N_PAD // NS

_BN = 2048
_GRID = N_PAD // _BN


def _tc1_body(x_ref, wl_ref, bl_ref, ws_ref, wn_ref, s_ref, q_ref):
    h = jnp.dot(x_ref[...], wl_ref[...], preferred_element_type=jnp.float32)
    h = h + bl_ref[...][None, :]
    s_ref[...] = jnp.dot(h, ws_ref[...], preferred_element_type=jnp.float32)
    q_ref[...] = jnp.dot(h, wn_ref[...], preferred_element_type=jnp.float32)


_tc1 = pl.pallas_call(
    _tc1_body,
    grid=(_GRID,),
    in_specs=[
        pl.BlockSpec((_BN, D), lambda i: (i, 0)),
        pl.BlockSpec((D, C), lambda i: (0, 0)),
        pl.BlockSpec((C,), lambda i: (0,)),
        pl.BlockSpec((C, H), lambda i: (0, 0)),
        pl.BlockSpec((C, H), lambda i: (0, 0)),
    ],
    out_specs=[
        pl.BlockSpec((_BN, H), lambda i: (i, 0)),
        pl.BlockSpec((_BN, H), lambda i: (i, 0)),
    ],
    out_shape=[
        jax.ShapeDtypeStruct((N_PAD, H), jnp.float32),
        jax.ShapeDtypeStruct((N_PAD, H), jnp.float32),
    ],
)


NBUF = 8
PREF = 6
CH0 = 112
CH1 = 48


def _make_sc_pass(with_deg: bool):
    out_type = [jax.ShapeDtypeStruct((NC, N_PAD, H), jnp.float32)]
    scratch = [
        pltpu.VMEM((CH0, CHUNK), jnp.int32),
        pltpu.VMEM((CH0, CHUNK), jnp.int32),
        pltpu.VMEM((NBUF, CHUNK, H), jnp.float32),
        pltpu.VMEM_SHARED((N_PAD, H), jnp.float32),
        [pltpu.SemaphoreType.DMA] * NBUF,
        [pltpu.SemaphoreType.DMA] * NBUF,
    ]
    if with_deg:
        out_type.append(jax.ShapeDtypeStruct((NC, N_PAD), jnp.float32))
        scratch.append(pltpu.VMEM((CHUNK,), jnp.float32))
        scratch.append(pltpu.VMEM_SHARED((N_PAD,), jnp.float32))

    def body(table, src_idx, dst_idx, z16, zd, *rest):
        if with_deg:
            acc_out, deg_out, srcv, dstv, rows, acc_sh, sem_g, sem_s, ones_v, deg_sh = rest
        else:
            acc_out, srcv, dstv, rows, acc_sh, sem_g, sem_s = rest
        cid = lax.axis_index("c")
        sid = lax.axis_index("s")
        start = sid * (CH0 + CH1) + cid * CH0
        chn = jnp.where(cid == 0, CH0, CH1)
        sl = pl.ds(sid * RPT, RPT)

        @pl.when(cid == 0)
        def _():
            pltpu.sync_copy(src_idx.at[pl.ds(start, CH0)], srcv)
            pltpu.sync_copy(dst_idx.at[pl.ds(start, CH0)], dstv)

        @pl.when(cid == 1)
        def _():
            pltpu.sync_copy(src_idx.at[pl.ds(start, CH1)], srcv.at[pl.ds(0, CH1)])
            pltpu.sync_copy(dst_idx.at[pl.ds(start, CH1)], dstv.at[pl.ds(0, CH1)])

        pltpu.sync_copy(z16.at[sl], acc_sh.at[sl])
        if with_deg:
            pltpu.sync_copy(zd.at[sl], deg_sh.at[sl])
            for i in range(CHUNK // 16):
                ones_v[pl.ds(i * 16, 16)] = jnp.full((16,), 1.0, jnp.float32)
        plsc.subcore_barrier()

        def wait_scatter(b, c):
            pltpu.make_async_copy(rows.at[b], acc_sh.at[dstv.at[c]], sem_s[b]).wait()
            if with_deg:
                pltpu.make_async_copy(ones_v, deg_sh.at[dstv.at[c]], sem_s[b]).wait()

        for b in range(PREF):
            pltpu.async_copy(table.at[srcv.at[b]], rows.at[b], sem_g[b])

        def step(i, carry):
            for b8 in range(NBUF):
                c = NBUF * i + b8
                gb = (b8 + PREF) % NBUF

                @pl.when(c + PREF < chn)
                def _():
                    @pl.when(c >= NBUF - PREF)
                    def _():
                        wait_scatter(gb, c + PREF - NBUF)

                    pltpu.async_copy(table.at[srcv.at[c + PREF]], rows.at[gb], sem_g[gb])

                pltpu.make_async_copy(table.at[srcv.at[c]], rows.at[b8], sem_g[b8]).wait()
                pltpu.async_copy(rows.at[b8], acc_sh.at[dstv.at[c]], sem_s[b8], add=True)
                if with_deg:
                    pltpu.async_copy(ones_v, deg_sh.at[dstv.at[c]], sem_s[b8], add=True)
            return carry

        lax.fori_loop(0, chn // NBUF, step, 0)
        for b in range(NBUF):
            wait_scatter(b, chn - NBUF + b)

        plsc.subcore_barrier()
        pltpu.sync_copy(acc_sh.at[sl], acc_out.at[cid, sl])
        if with_deg:
            pltpu.sync_copy(deg_sh.at[sl], deg_out.at[cid, sl])

    return pl.kernel(
        body,
        out_type=out_type,
        mesh=plsc.VectorSubcoreMesh(core_axis_name="c", subcore_axis_name="s"),
        scratch_types=scratch,
        compiler_params=pltpu.CompilerParams(use_tc_tiling_on_sc=False),
    )


_sc_pass_deg = _make_sc_pass(True)
_sc_pass = _make_sc_pass(False)


def _tc2_body(s_ref, acc_ref, deg_ref, b1_ref, h1_ref):
    a = acc_ref[0] + acc_ref[1]
    dg = deg_ref[0] + deg_ref[1]
    maxd = jnp.maximum(dg, 1.0)[:, None]
    h1_ref[...] = jnp.tanh(s_ref[...] + b1_ref[...][None, :] + a / maxd)


_tc2 = pl.pallas_call(
    _tc2_body,
    grid=(_GRID,),
    in_specs=[
        pl.BlockSpec((_BN, H), lambda i: (i, 0)),
        pl.BlockSpec((NC, _BN, H), lambda i: (0, i, 0)),
        pl.BlockSpec((NC, _BN), lambda i: (0, i)),
        pl.BlockSpec((H,), lambda i: (0,)),
    ],
    out_specs=pl.BlockSpec((_BN, H), lambda i: (i, 0)),
    out_shape=jax.ShapeDtypeStruct((N_PAD, H), jnp.float32),
)


def _tc3_body(h1_ref, acc_ref, deg_ref, ws_ref, wn_ref, b2_ref, out_ref):
    a = acc_ref[0] + acc_ref[1]
    dg = deg_ref[0] + deg_ref[1]
    agg = a / jnp.maximum(dg, 1.0)[:, None]
    out_ref[...] = jnp.tanh(
        jnp.dot(h1_ref[...], ws_ref[...], preferred_element_type=jnp.float32)
        + jnp.dot(agg, wn_ref[...], preferred_element_type=jnp.float32)
        + b2_ref[...][None, :]
    )


_tc3 = pl.pallas_call(
    _tc3_body,
    grid=(_GRID,),
    in_specs=[
        pl.BlockSpec((_BN, H), lambda i: (i, 0)),
        pl.BlockSpec((NC, _BN, H), lambda i: (0, i, 0)),
        pl.BlockSpec((NC, _BN), lambda i: (0, i)),
        pl.BlockSpec((H, C), lambda i: (0, 0)),
        pl.BlockSpec((H, C), lambda i: (0, 0)),
        pl.BlockSpec((C,), lambda i: (0,)),
    ],
    out_specs=pl.BlockSpec((_BN, C), lambda i: (i, 0)),
    out_shape=jax.ShapeDtypeStruct((N_PAD, C), jnp.float32),
)


def kernel(x, edge_index, W_lin, b_lin, W1_self, W1_neigh, b1, W2_self, W2_neigh, b2):
    x_pad = jnp.pad(x, ((0, N_PAD - N), (0, 0)))
    src = jnp.pad(edge_index[0], (0, E_PAD - E), constant_values=N)
    dst = jnp.pad(edge_index[1], (0, E_PAD - E), constant_values=N)
    src_r = src.reshape(E_PAD // CHUNK, CHUNK)
    dst_r = dst.reshape(E_PAD // CHUNK, CHUNK)
    z16 = jnp.zeros((N_PAD, H), jnp.float32)
    zd = jnp.zeros((N_PAD,), jnp.float32)

    s_tab, q_tab = _tc1(x_pad, W_lin, b_lin, W1_self, W1_neigh)
    acc1, deg = _sc_pass_deg(q_tab, src_r, dst_r, z16, zd)
    h1 = _tc2(s_tab, acc1, deg, b1)
    (acc2,) = _sc_pass(h1, src_r, dst_r, z16, zd)
    out = _tc3(h1, acc2, deg, W2_self, W2_neigh, b2)
    return out[:N]


# speedup vs baseline: 19.0175x; 1.0334x over previous
import jax
import jax.numpy as jnp
from jax import lax
from jax.experimental import pallas as pl
from jax.experimental.pallas import tpu as pltpu
from jax.experimental.pallas import tpu_sc as plsc

N = 10000
E = 320000
D = 128
H = 16
C = 128

NC = 2
NS = 16
CHUNK = 128
NCHUNKS = 2560
E_PAD = NCHUNKS * CHUNK
N_PAD = 10240
RPT = N_PAD // NS

NBUF = 8
PREF = 6
CH0 = 112
CH1 = 48

_BN = 2048
_BP = _BN // 8
_GRID = N_PAD // _BN


def _tc1_body(x_ref, wl_ref, bl_ref, ws_ref, wn_ref, s_ref, q_ref):
    xb = x_ref[...].reshape(8 * _BP, D)
    h = jnp.dot(xb, wl_ref[...], preferred_element_type=jnp.float32)
    h = h + bl_ref[...][None, :]
    s = jnp.dot(h, ws_ref[...], preferred_element_type=jnp.float32)
    q = jnp.dot(h, wn_ref[...], preferred_element_type=jnp.float32)
    s_ref[...] = jnp.concatenate([s[j * _BP:(j + 1) * _BP] for j in range(8)], axis=1)
    q_ref[...] = jnp.concatenate([q[j * _BP:(j + 1) * _BP] for j in range(8)], axis=1)


_tc1 = pl.pallas_call(
    _tc1_body,
    grid=(_GRID,),
    in_specs=[
        pl.BlockSpec((8, _BP, D), lambda i: (0, i, 0)),
        pl.BlockSpec((D, C), lambda i: (0, 0)),
        pl.BlockSpec((C,), lambda i: (0,)),
        pl.BlockSpec((C, H), lambda i: (0, 0)),
        pl.BlockSpec((C, H), lambda i: (0, 0)),
    ],
    out_specs=[
        pl.BlockSpec((_BP, 128), lambda i: (i, 0)),
        pl.BlockSpec((_BP, 128), lambda i: (i, 0)),
    ],
    out_shape=[
        jax.ShapeDtypeStruct((N_PAD // 8, 128), jnp.float32),
        jax.ShapeDtypeStruct((N_PAD // 8, 128), jnp.float32),
    ],
)


def _zero_shared(zbuf, shared, sid):
    def zfill(i, carry):
        zbuf[i, :] = jnp.zeros((H,), jnp.float32)
        return carry

    lax.fori_loop(0, CHUNK, zfill, 0)
    for k in range(RPT // CHUNK):
        pltpu.sync_copy(zbuf, shared.at[pl.ds(sid * RPT + k * CHUNK, CHUNK)])


def _sc_deg_body(dst_idx, deg_out, dstv, ones_v, zbuf, deg_sh, *sems):
    cid = lax.axis_index("c")
    sid = lax.axis_index("s")
    start = sid * (CH0 + CH1) + cid * (CH0 + CH1) // 2
    chn = (CH0 + CH1) // 2
    sl = pl.ds(sid * RPT, RPT)

    pltpu.sync_copy(dst_idx.at[pl.ds(start, chn)], dstv)

    def ofill(i, carry):
        ones_v[i, :] = jnp.full((H,), 1.0, jnp.float32)
        return carry

    lax.fori_loop(0, CHUNK, ofill, 0)
    _zero_shared(zbuf, deg_sh, sid)
    plsc.subcore_barrier()

    def step(i, carry):
        for b8 in range(NBUF):
            c = NBUF * i + b8

            @pl.when(c >= NBUF)
            def _():
                pltpu.make_async_copy(ones_v, deg_sh.at[dstv.at[c - NBUF]], sems[b8]).wait()

            pltpu.async_copy(ones_v, deg_sh.at[dstv.at[c]], sems[b8], add=True)
        return carry

    lax.fori_loop(0, chn // NBUF, step, 0)
    for b in range(NBUF):
        pltpu.make_async_copy(ones_v, deg_sh.at[dstv.at[chn - NBUF + b]], sems[b]).wait()

    plsc.subcore_barrier()
    pltpu.sync_copy(deg_sh.at[sl], deg_out.at[cid, sl])


_sc_deg = pl.kernel(
    _sc_deg_body,
    out_type=[jax.ShapeDtypeStruct((NC, N_PAD, H), jnp.float32)],
    mesh=plsc.VectorSubcoreMesh(core_axis_name="c", subcore_axis_name="s"),
    scratch_types=[
        pltpu.VMEM(((CH0 + CH1) // 2, CHUNK), jnp.int32),
        pltpu.VMEM((CHUNK, H), jnp.float32),
        pltpu.VMEM((CHUNK, H), jnp.float32),
        pltpu.VMEM_SHARED((N_PAD, H), jnp.float32),
    ]
    + [pltpu.SemaphoreType.DMA] * NBUF,
    compiler_params=pltpu.CompilerParams(use_tc_tiling_on_sc=False),
)


def _sc_pass_body(table, src_idx, dst_idx, acc_out, srcv, dstv, rows, zbuf,
                  acc_sh, *sems):
    sem_g, sem_s = sems[:NBUF], sems[NBUF:]
    cid = lax.axis_index("c")
    sid = lax.axis_index("s")
    start = sid * (CH0 + CH1) + cid * CH0
    chn = jnp.where(cid == 0, CH0, CH1)
    sl = pl.ds(sid * RPT, RPT)

    @pl.when(cid == 0)
    def _():
        pltpu.sync_copy(src_idx.at[pl.ds(start, CH0)], srcv)
        pltpu.sync_copy(dst_idx.at[pl.ds(start, CH0)], dstv)

    @pl.when(cid == 1)
    def _():
        pltpu.sync_copy(src_idx.at[pl.ds(start, CH1)], srcv.at[pl.ds(0, CH1)])
        pltpu.sync_copy(dst_idx.at[pl.ds(start, CH1)], dstv.at[pl.ds(0, CH1)])

    _zero_shared(zbuf, acc_sh, sid)
    plsc.subcore_barrier()

    def wait_scatter(b, c):
        pltpu.make_async_copy(rows.at[b], acc_sh.at[dstv.at[c]], sem_s[b]).wait()

    for b in range(PREF):
        pltpu.async_copy(table.at[srcv.at[b]], rows.at[b], sem_g[b])

    def step(i, carry):
        for b8 in range(NBUF):
            c = NBUF * i + b8
            gb = (b8 + PREF) % NBUF

            @pl.when(c + PREF < chn)
            def _():
                @pl.when(c >= NBUF - PREF)
                def _():
                    wait_scatter(gb, c + PREF - NBUF)

                pltpu.async_copy(table.at[srcv.at[c + PREF]], rows.at[gb], sem_g[gb])

            pltpu.make_async_copy(table.at[srcv.at[c]], rows.at[b8], sem_g[b8]).wait()
            pltpu.async_copy(rows.at[b8], acc_sh.at[dstv.at[c]], sem_s[b8], add=True)
        return carry

    lax.fori_loop(0, chn // NBUF, step, 0)
    for b in range(NBUF):
        wait_scatter(b, chn - NBUF + b)

    plsc.subcore_barrier()
    pltpu.sync_copy(acc_sh.at[sl], acc_out.at[cid, sl])


_sc_pass = pl.kernel(
    _sc_pass_body,
    out_type=[jax.ShapeDtypeStruct((NC, N_PAD, H), jnp.float32)],
    mesh=plsc.VectorSubcoreMesh(core_axis_name="c", subcore_axis_name="s"),
    scratch_types=[
        pltpu.VMEM((CH0, CHUNK), jnp.int32),
        pltpu.VMEM((CH0, CHUNK), jnp.int32),
        pltpu.VMEM((NBUF, CHUNK, H), jnp.float32),
        pltpu.VMEM((CHUNK, H), jnp.float32),
        pltpu.VMEM_SHARED((N_PAD, H), jnp.float32),
    ]
    + [pltpu.SemaphoreType.DMA] * (2 * NBUF),
    compiler_params=pltpu.CompilerParams(use_tc_tiling_on_sc=False),
)


def _tc2_body(s_ref, acc_ref, deg_ref, b1t_ref, h1_ref):
    a = acc_ref[0] + acc_ref[1]
    d = deg_ref[0] + deg_ref[1]
    h1_ref[...] = jnp.tanh(s_ref[...] + b1t_ref[...][None, :] + a / jnp.maximum(d, 1.0))


_tc2 = pl.pallas_call(
    _tc2_body,
    grid=(_GRID,),
    in_specs=[
        pl.BlockSpec((_BP, 128), lambda i: (i, 0)),
        pl.BlockSpec((NC, _BP, 128), lambda i: (0, i, 0)),
        pl.BlockSpec((NC, _BP, 128), lambda i: (0, i, 0)),
        pl.BlockSpec((128,), lambda i: (0,)),
    ],
    out_specs=pl.BlockSpec((_BP, 128), lambda i: (i, 0)),
    out_shape=jax.ShapeDtypeStruct((N_PAD // 8, 128), jnp.float32),
)


def _tc3_body(h1_ref, acc_ref, deg_ref, ws_ref, wn_ref, b2_ref, out_ref):
    hp = h1_ref[...]
    ap = (acc_ref[0] + acc_ref[1]) / jnp.maximum(deg_ref[0] + deg_ref[1], 1.0)
    for j in range(8):
        h1 = hp[:, j * H:(j + 1) * H]
        agg = ap[:, j * H:(j + 1) * H]
        out_ref[j] = jnp.tanh(
            jnp.dot(h1, ws_ref[...], preferred_element_type=jnp.float32)
            + jnp.dot(agg, wn_ref[...], preferred_element_type=jnp.float32)
            + b2_ref[...][None, :]
        )


_tc3 = pl.pallas_call(
    _tc3_body,
    grid=(_GRID,),
    in_specs=[
        pl.BlockSpec((_BP, 128), lambda i: (i, 0)),
        pl.BlockSpec((NC, _BP, 128), lambda i: (0, i, 0)),
        pl.BlockSpec((NC, _BP, 128), lambda i: (0, i, 0)),
        pl.BlockSpec((H, C), lambda i: (0, 0)),
        pl.BlockSpec((H, C), lambda i: (0, 0)),
        pl.BlockSpec((C,), lambda i: (0,)),
    ],
    out_specs=pl.BlockSpec((8, _BP, C), lambda i: (0, i, 0)),
    out_shape=jax.ShapeDtypeStruct((8, N_PAD // 8, C), jnp.float32),
)


def kernel(x, edge_index, W_lin, b_lin, W1_self, W1_neigh, b1, W2_self, W2_neigh, b2):
    x3 = jnp.pad(x, ((0, N_PAD - N), (0, 0))).reshape(8, N_PAD // 8, D)
    src = jnp.pad(edge_index[0], (0, E_PAD - E))
    dst = jnp.pad(edge_index[1], (0, E_PAD - E), constant_values=N)
    src_r = (8 * (src % (N_PAD // 8)) + src // (N_PAD // 8)).reshape(NCHUNKS, CHUNK)
    dst_r = (8 * (dst % (N_PAD // 8)) + dst // (N_PAD // 8)).reshape(NCHUNKS, CHUNK)
    b1t = jnp.tile(b1, 8)

    (deg,) = _sc_deg(dst_r)
    degp = deg.reshape(NC, N_PAD // 8, 128)
    sp, qp = _tc1(x3, W_lin, b_lin, W1_self, W1_neigh)
    (acc1,) = _sc_pass(qp.reshape(N_PAD, H), src_r, dst_r)
    h1p = _tc2(sp, acc1.reshape(NC, N_PAD // 8, 128), degp, b1t)
    (acc2,) = _sc_pass(h1p.reshape(N_PAD, H), src_r, dst_r)
    out = _tc3(h1p, acc2.reshape(NC, N_PAD // 8, 128), degp, W2_self, W2_neigh, b2)
    return out.reshape(N_PAD, C)[:N]


# speedup vs baseline: 19.1854x; 1.0088x over previous
import jax
import jax.numpy as jnp
from jax import lax
from jax.experimental import pallas as pl
from jax.experimental.pallas import tpu as pltpu
from jax.experimental.pallas import tpu_sc as plsc

N = 10000
E = 320000
D = 128
H = 16
C = 128

NC = 2
NS = 16
CHUNK = 128
NCHUNKS = 2560
E_PAD = NCHUNKS * CHUNK
N_PAD = 10240
RPT = N_PAD // NS

NBUF = 8
PREF = 6
CHW = 80

_BN = 2048
_BP = _BN // 8
_GRID = N_PAD // _BN


def _tc1_body(x_ref, wl_ref, bl_ref, ws_ref, wn_ref, s_ref, q_ref):
    xb = x_ref[...].reshape(8 * _BP, D)
    h = jnp.dot(xb, wl_ref[...], preferred_element_type=jnp.float32)
    h = h + bl_ref[...][None, :]
    s = jnp.dot(h, ws_ref[...], preferred_element_type=jnp.float32)
    q = jnp.dot(h, wn_ref[...], preferred_element_type=jnp.float32)
    s_ref[...] = jnp.concatenate([s[j * _BP:(j + 1) * _BP] for j in range(8)], axis=1)
    q_ref[...] = jnp.concatenate([q[j * _BP:(j + 1) * _BP] for j in range(8)], axis=1)


_tc1 = pl.pallas_call(
    _tc1_body,
    grid=(_GRID,),
    in_specs=[
        pl.BlockSpec((8, _BP, D), lambda i: (0, i, 0)),
        pl.BlockSpec((D, C), lambda i: (0, 0)),
        pl.BlockSpec((C,), lambda i: (0,)),
        pl.BlockSpec((C, H), lambda i: (0, 0)),
        pl.BlockSpec((C, H), lambda i: (0, 0)),
    ],
    out_specs=[
        pl.BlockSpec((_BP, 128), lambda i: (i, 0)),
        pl.BlockSpec((_BP, 128), lambda i: (i, 0)),
    ],
    out_shape=[
        jax.ShapeDtypeStruct((N_PAD // 8, 128), jnp.float32),
        jax.ShapeDtypeStruct((N_PAD // 8, 128), jnp.float32),
    ],
)


def _zero_shared(zbuf, shared, sid):
    def zfill(i, carry):
        zbuf[i, :] = jnp.zeros((H,), jnp.float32)
        return carry

    lax.fori_loop(0, CHUNK, zfill, 0)
    for k in range(RPT // CHUNK):
        pltpu.sync_copy(zbuf, shared.at[pl.ds(sid * RPT + k * CHUNK, CHUNK)])


def _make_sc_pass(with_deg: bool):
    out_type = [jax.ShapeDtypeStruct((NC, N_PAD, H), jnp.float32)]
    scratch = [
        pltpu.VMEM((CHW, CHUNK), jnp.int32),
        pltpu.VMEM((CHW, CHUNK), jnp.int32),
        pltpu.VMEM((NBUF, CHUNK, H), jnp.float32),
        pltpu.VMEM((CHUNK, H), jnp.float32),
        pltpu.VMEM_SHARED((N_PAD, H), jnp.float32),
    ]
    if with_deg:
        out_type.append(jax.ShapeDtypeStruct((NC, N_PAD, H), jnp.float32))
        scratch.append(pltpu.VMEM((CHUNK, H), jnp.float32))
        scratch.append(pltpu.VMEM_SHARED((N_PAD, H), jnp.float32))

    def body(table, src_idx, dst_idx, *rest):
        if with_deg:
            (acc_out, deg_out, srcv, dstv, rows, zbuf, acc_sh,
             ones_v, deg_sh) = rest[:9]
            sems = rest[9:]
        else:
            acc_out, srcv, dstv, rows, zbuf, acc_sh = rest[:6]
            sems = rest[6:]
        sem_g, sem_s = sems[:NBUF], sems[NBUF:2 * NBUF]
        sem_d = sems[2 * NBUF:]
        cid = lax.axis_index("c")
        sid = lax.axis_index("s")
        start = sid * (2 * CHW) + cid * CHW
        sl = pl.ds(sid * RPT, RPT)

        pltpu.sync_copy(src_idx.at[pl.ds(start, CHW)], srcv)
        pltpu.sync_copy(dst_idx.at[pl.ds(start, CHW)], dstv)

        _zero_shared(zbuf, acc_sh, sid)
        if with_deg:
            _zero_shared(zbuf, deg_sh, sid)

            def ofill(i, carry):
                ones_v[i, :] = jnp.full((H,), 1.0, jnp.float32)
                return carry

            lax.fori_loop(0, CHUNK, ofill, 0)
        plsc.subcore_barrier()

        def wait_scatter(b, c):
            pltpu.make_async_copy(rows.at[b], acc_sh.at[dstv.at[c]], sem_s[b]).wait()
            if with_deg:
                pltpu.make_async_copy(ones_v, deg_sh.at[dstv.at[c]], sem_d[b]).wait()

        for b in range(PREF):
            pltpu.async_copy(table.at[srcv.at[b]], rows.at[b], sem_g[b])

        def step(i, carry):
            for b8 in range(NBUF):
                c = NBUF * i + b8
                gb = (b8 + PREF) % NBUF

                @pl.when(c + PREF < CHW)
                def _():
                    @pl.when(c >= NBUF - PREF)
                    def _():
                        wait_scatter(gb, c + PREF - NBUF)

                    pltpu.async_copy(table.at[srcv.at[c + PREF]], rows.at[gb], sem_g[gb])

                pltpu.make_async_copy(table.at[srcv.at[c]], rows.at[b8], sem_g[b8]).wait()
                pltpu.async_copy(rows.at[b8], acc_sh.at[dstv.at[c]], sem_s[b8], add=True)
                if with_deg:
                    pltpu.async_copy(ones_v, deg_sh.at[dstv.at[c]], sem_d[b8], add=True)
            return carry

        lax.fori_loop(0, CHW // NBUF, step, 0)
        for b in range(NBUF):
            wait_scatter(b, CHW - NBUF + b)

        plsc.subcore_barrier()
        pltpu.sync_copy(acc_sh.at[sl], acc_out.at[cid, sl])
        if with_deg:
            pltpu.sync_copy(deg_sh.at[sl], deg_out.at[cid, sl])

    n_sems = 3 * NBUF if with_deg else 2 * NBUF
    return pl.kernel(
        body,
        out_type=out_type,
        mesh=plsc.VectorSubcoreMesh(core_axis_name="c", subcore_axis_name="s"),
        scratch_types=scratch + [pltpu.SemaphoreType.DMA] * n_sems,
        compiler_params=pltpu.CompilerParams(use_tc_tiling_on_sc=False),
    )


_sc_pass_deg = _make_sc_pass(True)
_sc_pass = _make_sc_pass(False)


def _tc2_body(s_ref, acc_ref, deg_ref, b1t_ref, h1_ref):
    a = acc_ref[0] + acc_ref[1]
    d = deg_ref[0] + deg_ref[1]
    h1_ref[...] = jnp.tanh(s_ref[...] + b1t_ref[...][None, :] + a / jnp.maximum(d, 1.0))


_tc2 = pl.pallas_call(
    _tc2_body,
    grid=(_GRID,),
    in_specs=[
        pl.BlockSpec((_BP, 128), lambda i: (i, 0)),
        pl.BlockSpec((NC, _BP, 128), lambda i: (0, i, 0)),
        pl.BlockSpec((NC, _BP, 128), lambda i: (0, i, 0)),
        pl.BlockSpec((128,), lambda i: (0,)),
    ],
    out_specs=pl.BlockSpec((_BP, 128), lambda i: (i, 0)),
    out_shape=jax.ShapeDtypeStruct((N_PAD // 8, 128), jnp.float32),
)


def _tc3_body(h1_ref, acc_ref, deg_ref, ws_ref, wn_ref, b2_ref, out_ref):
    hp = h1_ref[...]
    ap = (acc_ref[0] + acc_ref[1]) / jnp.maximum(deg_ref[0] + deg_ref[1], 1.0)
    for j in range(8):
        h1 = hp[:, j * H:(j + 1) * H]
        agg = ap[:, j * H:(j + 1) * H]
        out_ref[j] = jnp.tanh(
            jnp.dot(h1, ws_ref[...], preferred_element_type=jnp.float32)
            + jnp.dot(agg, wn_ref[...], preferred_element_type=jnp.float32)
            + b2_ref[...][None, :]
        )


_tc3 = pl.pallas_call(
    _tc3_body,
    grid=(_GRID,),
    in_specs=[
        pl.BlockSpec((_BP, 128), lambda i: (i, 0)),
        pl.BlockSpec((NC, _BP, 128), lambda i: (0, i, 0)),
        pl.BlockSpec((NC, _BP, 128), lambda i: (0, i, 0)),
        pl.BlockSpec((H, C), lambda i: (0, 0)),
        pl.BlockSpec((H, C), lambda i: (0, 0)),
        pl.BlockSpec((C,), lambda i: (0,)),
    ],
    out_specs=pl.BlockSpec((8, _BP, C), lambda i: (0, i, 0)),
    out_shape=jax.ShapeDtypeStruct((8, N_PAD // 8, C), jnp.float32),
)


def kernel(x, edge_index, W_lin, b_lin, W1_self, W1_neigh, b1, W2_self, W2_neigh, b2):
    x3 = jnp.pad(x, ((0, N_PAD - N), (0, 0))).reshape(8, N_PAD // 8, D)
    src = jnp.pad(edge_index[0], (0, E_PAD - E))
    dst = jnp.pad(edge_index[1], (0, E_PAD - E), constant_values=N)
    src_r = (8 * (src % (N_PAD // 8)) + src // (N_PAD // 8)).reshape(NCHUNKS, CHUNK)
    dst_r = (8 * (dst % (N_PAD // 8)) + dst // (N_PAD // 8)).reshape(NCHUNKS, CHUNK)
    b1t = jnp.tile(b1, 8)

    sp, qp = _tc1(x3, W_lin, b_lin, W1_self, W1_neigh)
    acc1, deg = _sc_pass_deg(qp.reshape(N_PAD, H), src_r, dst_r)
    acc1 = acc1.reshape(NC, N_PAD // 8, 128)
    degp = deg.reshape(NC, N_PAD // 8, 128)
    h1p = _tc2(sp, acc1, degp, b1t)
    (acc2,) = _sc_pass(h1p.reshape(N_PAD, H), src_r, dst_r)
    out = _tc3(h1p, acc2.reshape(NC, N_PAD // 8, 128), degp, W2_self, W2_neigh, b2)
    return out.reshape(N_PAD, C)[:N]
